# baseline (speedup 1.0000x reference)
import functools

import jax
import jax.numpy as jnp
from jax import lax
from jax.experimental import pallas as pl
from jax.experimental.pallas import tpu as pltpu
from jax.experimental.pallas import tpu_sc as plsc

B, N, M, K = 4, 8192, 1024, 32
C_IN, C_OUT, POS = 128, 256, 3

NSUB = 8
NLANE = N // NSUB


def _fps_kernel(xyzT_ref, out_ref):
    X = xyzT_ref[0]
    Y = xyzT_ref[1]
    Z = xyzT_ref[2]
    shp = (B, NSUB, NLANE)
    idx3 = (jax.lax.broadcasted_iota(jnp.int32, shp, 1) * NLANE
            + jax.lax.broadcasted_iota(jnp.int32, shp, 2))
    asub = jax.lax.broadcasted_iota(jnp.int32, (B, NSUB, M // NSUB), 1)
    alane = jax.lax.broadcasted_iota(jnp.int32, (B, NSUB, M // NSUB), 2)

    def body(i, carry):
        dists, far, cx, cy, cz, acc = carry
        acc = jnp.where((asub == i // (M // NSUB)) & (alane == i % (M // NSUB)),
                        far, acc)
        dx = X - cx
        dy = Y - cy
        dz = Z - cz
        d = dx * dx + dy * dy + dz * dz
        dists = jnp.minimum(dists, d)
        v, ii, tx, ty, tz = dists, idx3, X, Y, Z
        w = NLANE
        while w > 128:
            h = w // 2
            cond = v[..., :h] >= v[..., h:w]
            v = jnp.where(cond, v[..., :h], v[..., h:w])
            ii = jnp.where(cond, ii[..., :h], ii[..., h:w])
            tx = jnp.where(cond, tx[..., :h], tx[..., h:w])
            ty = jnp.where(cond, ty[..., :h], ty[..., h:w])
            tz = jnp.where(cond, tz[..., :h], tz[..., h:w])
            w = h
        s = NSUB
        while s > 1:
            h = s // 2
            cond = v[:, :h] >= v[:, h:s]
            v = jnp.where(cond, v[:, :h], v[:, h:s])
            ii = jnp.where(cond, ii[:, :h], ii[:, h:s])
            tx = jnp.where(cond, tx[:, :h], tx[:, h:s])
            ty = jnp.where(cond, ty[:, :h], ty[:, h:s])
            tz = jnp.where(cond, tz[:, :h], tz[:, h:s])
            s = h
        mx = jnp.max(v, axis=2, keepdims=True)
        m2 = v == mx
        far = jnp.min(jnp.where(m2, ii, N), axis=2, keepdims=True)
        m3 = ii == far
        cx = jnp.sum(jnp.where(m3, tx, 0.0), axis=2, keepdims=True)
        cy = jnp.sum(jnp.where(m3, ty, 0.0), axis=2, keepdims=True)
        cz = jnp.sum(jnp.where(m3, tz, 0.0), axis=2, keepdims=True)
        return dists, far, cx, cy, cz, acc

    dists0 = jnp.full(shp, 1e10, jnp.float32)
    far0 = jnp.zeros((B, 1, 1), jnp.int32)
    acc0 = jnp.zeros((B, NSUB, M // NSUB), jnp.int32)
    cx0 = X[:, 0:1, 0:1]
    cy0 = Y[:, 0:1, 0:1]
    cz0 = Z[:, 0:1, 0:1]
    out = jax.lax.fori_loop(0, M, body,
                            (dists0, far0, cx0, cy0, cz0, acc0))
    out_ref[...] = out[5].reshape(B, M)


@jax.jit
def _fps(xyz):
    xyzT = xyz.transpose(2, 0, 1).reshape(3, B, NSUB, NLANE)
    return pl.pallas_call(
        _fps_kernel,
        out_shape=jax.ShapeDtypeStruct((B, M), jnp.int32),
    )(xyzT)


NT = 2048
F_IN = POS + C_IN


def _pre_kernel(emb_ref, feat_ref, w1a_ref, w1b_ref, b1_ref, out_ref):
    e = emb_ref[0]
    f = feat_ref[0]
    out_ref[0] = (jnp.dot(f, w1b_ref[...], preferred_element_type=jnp.float32)
                  + jnp.dot(e, w1a_ref[...], preferred_element_type=jnp.float32)
                  + b1_ref[0, :])


@jax.jit
def _pre(xyz_embed, features, W1a, W1b, b1):
    return pl.pallas_call(
        _pre_kernel,
        grid=(B, N // NT),
        in_specs=[
            pl.BlockSpec((1, NT, POS), lambda b, i: (b, i, 0)),
            pl.BlockSpec((1, NT, C_IN), lambda b, i: (b, i, 0)),
            pl.BlockSpec((POS, C_OUT), lambda b, i: (0, 0)),
            pl.BlockSpec((C_IN, C_OUT), lambda b, i: (0, 0)),
            pl.BlockSpec((1, C_OUT), lambda b, i: (0, 0)),
        ],
        out_specs=pl.BlockSpec((1, NT, C_OUT), lambda b, i: (b, i, 0)),
        out_shape=jax.ShapeDtypeStruct((B, N, C_OUT), jnp.float32),
    )(xyz_embed, features, W1a, W1b, b1)


TM = 128
NBIN = 64
BINSZ = N // NBIN
ROUNDS = 8
IMAX = 2**31 - 1


def _knn_kernel(xyz_ref, cT_ref, out_ref):
    x = xyz_ref[0]
    cT = cT_ref[0]
    dot = jnp.dot(x, cT, preferred_element_type=jnp.float32)
    xn2 = jnp.sum(x * x, axis=1, keepdims=True)
    cn2 = jnp.sum(cT * cT, axis=0, keepdims=True)
    d2 = xn2 + cn2 - 2.0 * dot

    bits = jax.lax.bitcast_convert_type(d2, jnp.int32)
    bits3 = bits.reshape(NBIN, BINSZ, TM)
    s7 = jax.lax.broadcasted_iota(jnp.int32, (NBIN, BINSZ, TM), 1)
    P = (bits3 & jnp.int32(~127)) | s7

    cands = []
    for _ in range(ROUNDS):
        mt = jnp.min(P, axis=1)
        cands.append(mt)
        P = jnp.where(P == mt[:, None, :], IMAX, P)
    C0 = jnp.concatenate(cands, axis=0)

    NC = ROUNDS * NBIN
    sC = jax.lax.broadcasted_iota(jnp.int32, (NC, TM), 0)
    kio = jax.lax.broadcasted_iota(jnp.int32, (K, TM), 0)

    def body(k, carry):
        C, outp = carry
        mn = jnp.min(C, axis=0, keepdims=True)
        am = jnp.min(jnp.where(C == mn, sC, IMAX), axis=0,
                     keepdims=True)
        gidx = ((am & (NBIN - 1)) << 7) | (mn & 127)
        outp = jnp.where(kio == k, gidx, outp)
        C = jnp.where(sC == am, IMAX, C)
        return C, outp

    _, outp = jax.lax.fori_loop(0, K, body,
                                (C0, jnp.zeros((K, TM), jnp.int32)))
    out_ref[0] = outp


@jax.jit
def _knn(xyz, centersT):
    knnT = pl.pallas_call(
        _knn_kernel,
        grid=(B, M // TM),
        in_specs=[
            pl.BlockSpec((1, N, POS), lambda b, i: (b, 0, 0)),
            pl.BlockSpec((1, POS, TM), lambda b, i: (b, 0, i)),
        ],
        out_specs=pl.BlockSpec((1, K, TM), lambda b, i: (b, 0, i)),
        out_shape=jax.ShapeDtypeStruct((B, K, M), jnp.int32),
    )(xyz, centersT)
    return knnT


NWORK = 32
ROWS_TOT = B * M * K
RPW = ROWS_TOT // NWORK
GCH = 128
NCHUNK = RPW // GCH


@functools.lru_cache(maxsize=1)
def _gather_sc_fn():
    @functools.partial(
        pl.kernel,
        mesh=plsc.VectorSubcoreMesh(core_axis_name="c", subcore_axis_name="s"),
        out_type=jax.ShapeDtypeStruct((ROWS_TOT, C_OUT), jnp.float32),
        scratch_types=[
            pltpu.VMEM((NCHUNK, GCH), jnp.int32),
            pltpu.VMEM((GCH, C_OUT), jnp.float32),
            pltpu.VMEM((GCH, C_OUT), jnp.float32),
            pltpu.SemaphoreType.DMA,
            pltpu.SemaphoreType.DMA,
        ],
    )
    def _gather_sc(table_hbm, gidx_hbm, out_hbm, idx_all, rows0, rows1,
                   s0, s1):
        wid = lax.axis_index("s") * 2 + lax.axis_index("c")
        base = wid * RPW
        pltpu.sync_copy(gidx_hbm.at[wid], idx_all)
        rows = (rows0, rows1)
        sems = (s0, s1)
        pend = pltpu.async_copy(table_hbm.at[idx_all.at[0]], rows0, s0)
        for j in range(1, NCHUNK + 1):
            nxt = None
            if j < NCHUNK:
                nxt = pltpu.async_copy(table_hbm.at[idx_all.at[j]],
                                       rows[j % 2], sems[j % 2])
            pend.wait()
            pltpu.sync_copy(rows[(j - 1) % 2],
                            out_hbm.at[pl.ds(base + (j - 1) * GCH, GCH)])
            pend = nxt

    return _gather_sc


TC2 = 32
ROWS = TC2 * K


def _mlp_kernel(g_ref, ce1_ref, w2_ref, b2_ref, out_ref):
    g = g_ref[...].reshape(TC2, K, C_OUT)
    h1 = jnp.maximum(g - ce1_ref[...][:, None, :], 0.0).reshape(ROWS, C_OUT)
    h2 = jnp.maximum(
        jnp.dot(h1, w2_ref[...], preferred_element_type=jnp.float32)
        + b2_ref[0, :], 0.0)
    out_ref[...] = jnp.max(h2.reshape(TC2, K, C_OUT), axis=1)


@jax.jit
def _mlp_maxpool(g, ce1, W2, b2):
    rows = g.shape[0]
    return pl.pallas_call(
        _mlp_kernel,
        grid=(rows // ROWS,),
        in_specs=[
            pl.BlockSpec((ROWS, C_OUT), lambda i: (i, 0)),
            pl.BlockSpec((TC2, C_OUT), lambda i: (i, 0)),
            pl.BlockSpec((C_OUT, C_OUT), lambda i: (0, 0)),
            pl.BlockSpec((1, C_OUT), lambda i: (0, 0)),
        ],
        out_specs=pl.BlockSpec((TC2, C_OUT), lambda i: (i, 0)),
        out_shape=jax.ShapeDtypeStruct((rows // K, C_OUT), jnp.float32),
    )(g, ce1, W2, b2)


@jax.jit
def kernel(xyz, xyz_embed, features, W1, b1, W2, b2):
    b = xyz.shape[0]
    sample_ids = _fps(jax.lax.stop_gradient(xyz))
    bidx = jnp.arange(b)[:, None]
    centers = xyz[bidx, sample_ids]
    center_embed = xyz_embed[bidx, sample_ids]

    T = _pre(xyz_embed, features, W1[:POS, :], W1[POS:, :],
             b1.reshape(1, C_OUT))

    knnT = _knn(xyz, centers.transpose(0, 2, 1))
    knn_idx = knnT.transpose(0, 2, 1)

    gidx = (knn_idx + (jnp.arange(B, dtype=jnp.int32) * N)[:, None, None])
    G = _gather_sc_fn()(T.reshape(B * N, C_OUT),
                        gidx.reshape(NWORK, NCHUNK, GCH))
    ce1 = center_embed.reshape(-1, POS) @ W1[:POS, :]

    cf = _mlp_maxpool(G, ce1, W2, b2.reshape(1, C_OUT))
    center_features = cf.reshape(b, M, C_OUT)
    return (centers, center_embed, center_features, sample_ids)


# speedup vs baseline: 1.0143x; 1.0143x over previous
import functools

import jax
import jax.numpy as jnp
from jax import lax
from jax.experimental import pallas as pl
from jax.experimental.pallas import tpu as pltpu
from jax.experimental.pallas import tpu_sc as plsc

B, N, M, K = 4, 8192, 1024, 32
C_IN, C_OUT, POS = 128, 256, 3

NSUB = 8
NLANE = N // NSUB


def _fps_kernel(xyzT_ref, out_ref):
    X = xyzT_ref[0]
    Y = xyzT_ref[1]
    Z = xyzT_ref[2]
    shp = (B, NSUB, NLANE)
    idx3 = (jax.lax.broadcasted_iota(jnp.int32, shp, 1) * NLANE
            + jax.lax.broadcasted_iota(jnp.int32, shp, 2))
    asub = jax.lax.broadcasted_iota(jnp.int32, (B, NSUB, M // NSUB), 1)
    alane = jax.lax.broadcasted_iota(jnp.int32, (B, NSUB, M // NSUB), 2)

    def body(i, carry):
        dists, far, cx, cy, cz, acc = carry
        acc = jnp.where((asub == i // (M // NSUB)) & (alane == i % (M // NSUB)),
                        far, acc)
        dx = X - cx
        dy = Y - cy
        dz = Z - cz
        d = dx * dx + dy * dy + dz * dz
        dists = jnp.minimum(dists, d)
        v, ii, tx, ty, tz = dists, idx3, X, Y, Z
        w = NLANE
        while w > 128:
            h = w // 2
            cond = v[..., :h] >= v[..., h:w]
            v = jnp.where(cond, v[..., :h], v[..., h:w])
            ii = jnp.where(cond, ii[..., :h], ii[..., h:w])
            tx = jnp.where(cond, tx[..., :h], tx[..., h:w])
            ty = jnp.where(cond, ty[..., :h], ty[..., h:w])
            tz = jnp.where(cond, tz[..., :h], tz[..., h:w])
            w = h
        mx = jnp.max(v, axis=(1, 2), keepdims=True)
        m2 = v == mx
        far = jnp.min(jnp.where(m2, ii, N), axis=(1, 2), keepdims=True)
        m3 = ii == far
        cx = jnp.sum(jnp.where(m3, tx, 0.0), axis=(1, 2), keepdims=True)
        cy = jnp.sum(jnp.where(m3, ty, 0.0), axis=(1, 2), keepdims=True)
        cz = jnp.sum(jnp.where(m3, tz, 0.0), axis=(1, 2), keepdims=True)
        return dists, far, cx, cy, cz, acc

    dists0 = jnp.full(shp, 1e10, jnp.float32)
    far0 = jnp.zeros((B, 1, 1), jnp.int32)
    acc0 = jnp.zeros((B, NSUB, M // NSUB), jnp.int32)
    cx0 = X[:, 0:1, 0:1]
    cy0 = Y[:, 0:1, 0:1]
    cz0 = Z[:, 0:1, 0:1]
    out = jax.lax.fori_loop(0, M, body,
                            (dists0, far0, cx0, cy0, cz0, acc0))
    out_ref[...] = out[5].reshape(B, M)


@jax.jit
def _fps(xyz):
    xyzT = xyz.transpose(2, 0, 1).reshape(3, B, NSUB, NLANE)
    return pl.pallas_call(
        _fps_kernel,
        out_shape=jax.ShapeDtypeStruct((B, M), jnp.int32),
    )(xyzT)


NT = 2048
F_IN = POS + C_IN


def _pre_kernel(emb_ref, feat_ref, w1a_ref, w1b_ref, b1_ref, out_ref):
    e = emb_ref[0]
    f = feat_ref[0]
    out_ref[0] = (jnp.dot(f, w1b_ref[...], preferred_element_type=jnp.float32)
                  + jnp.dot(e, w1a_ref[...], preferred_element_type=jnp.float32)
                  + b1_ref[0, :])


@jax.jit
def _pre(xyz_embed, features, W1a, W1b, b1):
    return pl.pallas_call(
        _pre_kernel,
        grid=(B, N // NT),
        in_specs=[
            pl.BlockSpec((1, NT, POS), lambda b, i: (b, i, 0)),
            pl.BlockSpec((1, NT, C_IN), lambda b, i: (b, i, 0)),
            pl.BlockSpec((POS, C_OUT), lambda b, i: (0, 0)),
            pl.BlockSpec((C_IN, C_OUT), lambda b, i: (0, 0)),
            pl.BlockSpec((1, C_OUT), lambda b, i: (0, 0)),
        ],
        out_specs=pl.BlockSpec((1, NT, C_OUT), lambda b, i: (b, i, 0)),
        out_shape=jax.ShapeDtypeStruct((B, N, C_OUT), jnp.float32),
    )(xyz_embed, features, W1a, W1b, b1)


TM = 128
NBIN = 64
BINSZ = N // NBIN
ROUNDS = 8
IMAX = 2**31 - 1


def _knn_kernel(xyz_ref, cT_ref, out_ref):
    x = xyz_ref[0]
    cT = cT_ref[0]
    dot = jnp.dot(x, cT, preferred_element_type=jnp.float32)
    xn2 = jnp.sum(x * x, axis=1, keepdims=True)
    cn2 = jnp.sum(cT * cT, axis=0, keepdims=True)
    d2 = xn2 + cn2 - 2.0 * dot

    bits = jax.lax.bitcast_convert_type(d2, jnp.int32)
    bits3 = bits.reshape(NBIN, BINSZ, TM)
    s7 = jax.lax.broadcasted_iota(jnp.int32, (NBIN, BINSZ, TM), 1)
    P = (bits3 & jnp.int32(~127)) | s7

    cands = []
    for _ in range(ROUNDS):
        mt = jnp.min(P, axis=1)
        cands.append(mt)
        P = jnp.where(P == mt[:, None, :], IMAX, P)
    C0 = jnp.concatenate(cands, axis=0)

    NC = ROUNDS * NBIN
    sC = jax.lax.broadcasted_iota(jnp.int32, (NC, TM), 0)
    kio = jax.lax.broadcasted_iota(jnp.int32, (K, TM), 0)

    def body(k, carry):
        C, outp = carry
        mn = jnp.min(C, axis=0, keepdims=True)
        am = jnp.min(jnp.where(C == mn, sC, IMAX), axis=0,
                     keepdims=True)
        gidx = ((am & (NBIN - 1)) << 7) | (mn & 127)
        outp = jnp.where(kio == k, gidx, outp)
        C = jnp.where(sC == am, IMAX, C)
        return C, outp

    _, outp = jax.lax.fori_loop(0, K, body,
                                (C0, jnp.zeros((K, TM), jnp.int32)))
    out_ref[0] = outp


@jax.jit
def _knn(xyz, centersT):
    knnT = pl.pallas_call(
        _knn_kernel,
        grid=(B, M // TM),
        in_specs=[
            pl.BlockSpec((1, N, POS), lambda b, i: (b, 0, 0)),
            pl.BlockSpec((1, POS, TM), lambda b, i: (b, 0, i)),
        ],
        out_specs=pl.BlockSpec((1, K, TM), lambda b, i: (b, 0, i)),
        out_shape=jax.ShapeDtypeStruct((B, K, M), jnp.int32),
    )(xyz, centersT)
    return knnT


NWORK = 32
ROWS_TOT = B * M * K
RPW = ROWS_TOT // NWORK
GCH = 128
NCHUNK = RPW // GCH


@functools.lru_cache(maxsize=1)
def _gather_sc_fn():
    @functools.partial(
        pl.kernel,
        mesh=plsc.VectorSubcoreMesh(core_axis_name="c", subcore_axis_name="s"),
        out_type=jax.ShapeDtypeStruct((ROWS_TOT, C_OUT), jnp.float32),
        scratch_types=[
            pltpu.VMEM((NCHUNK, GCH), jnp.int32),
            pltpu.VMEM((GCH, C_OUT), jnp.float32),
            pltpu.VMEM((GCH, C_OUT), jnp.float32),
            pltpu.SemaphoreType.DMA,
            pltpu.SemaphoreType.DMA,
        ],
    )
    def _gather_sc(table_hbm, gidx_hbm, out_hbm, idx_all, rows0, rows1,
                   s0, s1):
        wid = lax.axis_index("s") * 2 + lax.axis_index("c")
        base = wid * RPW
        pltpu.sync_copy(gidx_hbm.at[wid], idx_all)
        rows = (rows0, rows1)
        sems = (s0, s1)
        pend = pltpu.async_copy(table_hbm.at[idx_all.at[0]], rows0, s0)
        for j in range(1, NCHUNK + 1):
            nxt = None
            if j < NCHUNK:
                nxt = pltpu.async_copy(table_hbm.at[idx_all.at[j]],
                                       rows[j % 2], sems[j % 2])
            pend.wait()
            pltpu.sync_copy(rows[(j - 1) % 2],
                            out_hbm.at[pl.ds(base + (j - 1) * GCH, GCH)])
            pend = nxt

    return _gather_sc


TC2 = 32
ROWS = TC2 * K


def _mlp_kernel(g_ref, ce1_ref, w2_ref, b2_ref, out_ref):
    g = g_ref[...].reshape(TC2, K, C_OUT)
    h1 = jnp.maximum(g - ce1_ref[...][:, None, :], 0.0).reshape(ROWS, C_OUT)
    h2 = jnp.maximum(
        jnp.dot(h1, w2_ref[...], preferred_element_type=jnp.float32)
        + b2_ref[0, :], 0.0)
    out_ref[...] = jnp.max(h2.reshape(TC2, K, C_OUT), axis=1)


@jax.jit
def _mlp_maxpool(g, ce1, W2, b2):
    rows = g.shape[0]
    return pl.pallas_call(
        _mlp_kernel,
        grid=(rows // ROWS,),
        in_specs=[
            pl.BlockSpec((ROWS, C_OUT), lambda i: (i, 0)),
            pl.BlockSpec((TC2, C_OUT), lambda i: (i, 0)),
            pl.BlockSpec((C_OUT, C_OUT), lambda i: (0, 0)),
            pl.BlockSpec((1, C_OUT), lambda i: (0, 0)),
        ],
        out_specs=pl.BlockSpec((TC2, C_OUT), lambda i: (i, 0)),
        out_shape=jax.ShapeDtypeStruct((rows // K, C_OUT), jnp.float32),
    )(g, ce1, W2, b2)


@jax.jit
def kernel(xyz, xyz_embed, features, W1, b1, W2, b2):
    b = xyz.shape[0]
    sample_ids = _fps(jax.lax.stop_gradient(xyz))
    bidx = jnp.arange(b)[:, None]
    centers = xyz[bidx, sample_ids]
    center_embed = xyz_embed[bidx, sample_ids]

    T = _pre(xyz_embed, features, W1[:POS, :], W1[POS:, :],
             b1.reshape(1, C_OUT))

    knnT = _knn(xyz, centers.transpose(0, 2, 1))
    knn_idx = knnT.transpose(0, 2, 1)

    gidx = (knn_idx + (jnp.arange(B, dtype=jnp.int32) * N)[:, None, None])
    G = _gather_sc_fn()(T.reshape(B * N, C_OUT),
                        gidx.reshape(NWORK, NCHUNK, GCH))
    ce1 = center_embed.reshape(-1, POS) @ W1[:POS, :]

    cf = _mlp_maxpool(G, ce1, W2, b2.reshape(1, C_OUT))
    center_features = cf.reshape(b, M, C_OUT)
    return (centers, center_embed, center_features, sample_ids)


# speedup vs baseline: 1.1013x; 1.0857x over previous
import functools

import jax
import jax.numpy as jnp
from jax import lax
from jax.experimental import pallas as pl
from jax.experimental.pallas import tpu as pltpu
from jax.experimental.pallas import tpu_sc as plsc

B, N, M, K = 4, 8192, 1024, 32
C_IN, C_OUT, POS = 128, 256, 3

NSUB = 8
NLANE = N // NSUB


def _fps_kernel(xyzT_ref, out_ref):
    X = xyzT_ref[0]
    Y = xyzT_ref[1]
    Z = xyzT_ref[2]
    shp = (B, NSUB, NLANE)
    idx3 = (jax.lax.broadcasted_iota(jnp.int32, shp, 1) * NLANE
            + jax.lax.broadcasted_iota(jnp.int32, shp, 2))
    asub = jax.lax.broadcasted_iota(jnp.int32, (B, NSUB, M // NSUB), 1)
    alane = jax.lax.broadcasted_iota(jnp.int32, (B, NSUB, M // NSUB), 2)

    def body(i, carry):
        dists, far, cx, cy, cz, acc = carry
        acc = jnp.where((asub == i // (M // NSUB)) & (alane == i % (M // NSUB)),
                        far, acc)
        dx = X - cx
        dy = Y - cy
        dz = Z - cz
        d = dx * dx + dy * dy + dz * dz
        dists = jnp.minimum(dists, d)
        v, ii, tx, ty, tz = dists, idx3, X, Y, Z
        w = NLANE
        while w > 128:
            h = w // 2
            cond = v[..., :h] >= v[..., h:w]
            v = jnp.where(cond, v[..., :h], v[..., h:w])
            ii = jnp.where(cond, ii[..., :h], ii[..., h:w])
            tx = jnp.where(cond, tx[..., :h], tx[..., h:w])
            ty = jnp.where(cond, ty[..., :h], ty[..., h:w])
            tz = jnp.where(cond, tz[..., :h], tz[..., h:w])
            w = h
        mx = jnp.max(v, axis=(1, 2), keepdims=True)
        m2 = v == mx
        far = jnp.min(jnp.where(m2, ii, N), axis=(1, 2), keepdims=True)
        m3 = ii == far
        cx = jnp.sum(jnp.where(m3, tx, 0.0), axis=(1, 2), keepdims=True)
        cy = jnp.sum(jnp.where(m3, ty, 0.0), axis=(1, 2), keepdims=True)
        cz = jnp.sum(jnp.where(m3, tz, 0.0), axis=(1, 2), keepdims=True)
        return dists, far, cx, cy, cz, acc

    dists0 = jnp.full(shp, 1e10, jnp.float32)
    far0 = jnp.zeros((B, 1, 1), jnp.int32)
    acc0 = jnp.zeros((B, NSUB, M // NSUB), jnp.int32)
    cx0 = X[:, 0:1, 0:1]
    cy0 = Y[:, 0:1, 0:1]
    cz0 = Z[:, 0:1, 0:1]
    out = jax.lax.fori_loop(0, M, body,
                            (dists0, far0, cx0, cy0, cz0, acc0))
    out_ref[...] = out[5].reshape(B, M)


@jax.jit
def _fps(xyz):
    xyzT = xyz.transpose(2, 0, 1).reshape(3, B, NSUB, NLANE)
    return pl.pallas_call(
        _fps_kernel,
        out_shape=jax.ShapeDtypeStruct((B, M), jnp.int32),
    )(xyzT)


NT = 2048
F_IN = POS + C_IN


def _pre_kernel(emb_ref, feat_ref, w1a_ref, w1b_ref, b1_ref, out_ref):
    e = emb_ref[0]
    f = feat_ref[0]
    out_ref[0] = (jnp.dot(f, w1b_ref[...], preferred_element_type=jnp.float32)
                  + jnp.dot(e, w1a_ref[...], preferred_element_type=jnp.float32)
                  + b1_ref[0, :])


@jax.jit
def _pre(xyz_embed, features, W1a, W1b, b1):
    return pl.pallas_call(
        _pre_kernel,
        grid=(B, N // NT),
        in_specs=[
            pl.BlockSpec((1, NT, POS), lambda b, i: (b, i, 0)),
            pl.BlockSpec((1, NT, C_IN), lambda b, i: (b, i, 0)),
            pl.BlockSpec((POS, C_OUT), lambda b, i: (0, 0)),
            pl.BlockSpec((C_IN, C_OUT), lambda b, i: (0, 0)),
            pl.BlockSpec((1, C_OUT), lambda b, i: (0, 0)),
        ],
        out_specs=pl.BlockSpec((1, NT, C_OUT), lambda b, i: (b, i, 0)),
        out_shape=jax.ShapeDtypeStruct((B, N, C_OUT), jnp.float32),
    )(xyz_embed, features, W1a, W1b, b1)


TM = 128
NBIN = 64
BINSZ = N // NBIN
ROUNDS = 6
IMAX = 2**31 - 1


def _knn_kernel(xyz_ref, cT_ref, out_ref):
    bno = pl.program_id(0) * N
    x = xyz_ref[0]
    cT = cT_ref[0]
    dot = jnp.dot(x, cT, preferred_element_type=jnp.float32)
    xn2 = jnp.sum(x * x, axis=1, keepdims=True)
    cn2 = jnp.sum(cT * cT, axis=0, keepdims=True)
    d2 = xn2 + cn2 - 2.0 * dot

    bits = jax.lax.bitcast_convert_type(d2, jnp.int32)
    bits3 = bits.reshape(NBIN, BINSZ, TM)
    s7 = jax.lax.broadcasted_iota(jnp.int32, (NBIN, BINSZ, TM), 1)
    P = (bits3 & jnp.int32(~127)) | s7

    cands = []
    for _ in range(ROUNDS):
        mt = jnp.min(P, axis=1)
        cands.append(mt)
        P = jnp.where(P == mt[:, None, :], IMAX, P)
    C0 = jnp.concatenate(cands, axis=0)

    NC = ROUNDS * NBIN
    sC = jax.lax.broadcasted_iota(jnp.int32, (NC, TM), 0)
    kio = jax.lax.broadcasted_iota(jnp.int32, (K, TM), 0)

    def body(k, carry):
        C, outp = carry
        mn = jnp.min(C, axis=0, keepdims=True)
        am = jnp.min(jnp.where(C == mn, sC, IMAX), axis=0,
                     keepdims=True)
        gidx = (((am & (NBIN - 1)) << 7) | (mn & 127)) + bno
        outp = jnp.where(kio == k, gidx, outp)
        C = jnp.where(sC == am, IMAX, C)
        return C, outp

    _, outp = jax.lax.fori_loop(0, K, body,
                                (C0, jnp.zeros((K, TM), jnp.int32)))
    out_ref[0] = outp


@jax.jit
def _knn(xyz, centersT):
    knnT = pl.pallas_call(
        _knn_kernel,
        grid=(B, M // TM),
        in_specs=[
            pl.BlockSpec((1, N, POS), lambda b, i: (b, 0, 0)),
            pl.BlockSpec((1, POS, TM), lambda b, i: (b, 0, i)),
        ],
        out_specs=pl.BlockSpec((1, K, TM), lambda b, i: (b, 0, i)),
        out_shape=jax.ShapeDtypeStruct((B, K, M), jnp.int32),
    )(xyz, centersT)
    return knnT


NWORK = 32
ROWS_TOT = B * M * K
RPW = ROWS_TOT // NWORK
GCH = 128
NCHUNK = RPW // GCH


@functools.lru_cache(maxsize=1)
def _gather_sc_fn():
    @functools.partial(
        pl.kernel,
        mesh=plsc.VectorSubcoreMesh(core_axis_name="c", subcore_axis_name="s"),
        out_type=jax.ShapeDtypeStruct((ROWS_TOT, C_OUT), jnp.float32),
        scratch_types=[
            pltpu.VMEM((NCHUNK, GCH), jnp.int32),
            pltpu.VMEM((GCH, C_OUT), jnp.float32),
            pltpu.VMEM((GCH, C_OUT), jnp.float32),
            pltpu.SemaphoreType.DMA,
            pltpu.SemaphoreType.DMA,
        ],
    )
    def _gather_sc(table_hbm, gidx_hbm, out_hbm, idx_all, rows0, rows1,
                   s0, s1):
        wid = lax.axis_index("s") * 2 + lax.axis_index("c")
        base = wid * RPW
        pltpu.sync_copy(gidx_hbm.at[wid], idx_all)
        rows = (rows0, rows1)
        sems = (s0, s1)
        pend = pltpu.async_copy(table_hbm.at[idx_all.at[0]], rows0, s0)
        for j in range(1, NCHUNK + 1):
            nxt = None
            if j < NCHUNK:
                nxt = pltpu.async_copy(table_hbm.at[idx_all.at[j]],
                                       rows[j % 2], sems[j % 2])
            pend.wait()
            pltpu.sync_copy(rows[(j - 1) % 2],
                            out_hbm.at[pl.ds(base + (j - 1) * GCH, GCH)])
            pend = nxt

    return _gather_sc


TC2 = 32
ROWS = TC2 * K


def _mlp_kernel(g_ref, ce1_ref, w2_ref, b2_ref, out_ref):
    g = g_ref[0]
    h1 = jnp.maximum(g - ce1_ref[0][None, :, :], 0.0).reshape(ROWS, C_OUT)
    h2 = jnp.maximum(
        jnp.dot(h1, w2_ref[...], preferred_element_type=jnp.float32)
        + b2_ref[0, :], 0.0)
    out_ref[0] = jnp.max(h2.reshape(K, TC2, C_OUT), axis=0)


@jax.jit
def _mlp_maxpool(g, ce1, W2, b2):
    return pl.pallas_call(
        _mlp_kernel,
        grid=(B, M // TC2),
        in_specs=[
            pl.BlockSpec((1, K, TC2, C_OUT), lambda b, i: (b, 0, i, 0)),
            pl.BlockSpec((1, TC2, C_OUT), lambda b, i: (b, i, 0)),
            pl.BlockSpec((C_OUT, C_OUT), lambda b, i: (0, 0)),
            pl.BlockSpec((1, C_OUT), lambda b, i: (0, 0)),
        ],
        out_specs=pl.BlockSpec((1, TC2, C_OUT), lambda b, i: (b, i, 0)),
        out_shape=jax.ShapeDtypeStruct((B, M, C_OUT), jnp.float32),
    )(g, ce1, W2, b2)


@jax.jit
def kernel(xyz, xyz_embed, features, W1, b1, W2, b2):
    b = xyz.shape[0]
    sample_ids = _fps(jax.lax.stop_gradient(xyz))
    bidx = jnp.arange(b)[:, None]
    centers = xyz[bidx, sample_ids]
    center_embed = xyz_embed[bidx, sample_ids]

    T = _pre(xyz_embed, features, W1[:POS, :], W1[POS:, :],
             b1.reshape(1, C_OUT))

    gidx = _knn(xyz, centers.transpose(0, 2, 1))
    G = _gather_sc_fn()(T.reshape(B * N, C_OUT),
                        gidx.reshape(NWORK, NCHUNK, GCH))
    ce1 = (center_embed.reshape(-1, POS) @ W1[:POS, :]).reshape(B, M, C_OUT)

    center_features = _mlp_maxpool(G.reshape(B, K, M, C_OUT), ce1, W2,
                                   b2.reshape(1, C_OUT))
    return (centers, center_embed, center_features, sample_ids)


# speedup vs baseline: 1.1077x; 1.0058x over previous
import functools

import jax
import jax.numpy as jnp
from jax import lax
from jax.experimental import pallas as pl
from jax.experimental.pallas import tpu as pltpu
from jax.experimental.pallas import tpu_sc as plsc

B, N, M, K = 4, 8192, 1024, 32
C_IN, C_OUT, POS = 128, 256, 3

NSUB = 8
NLANE = N // NSUB


def _fps_kernel(xyzT_ref, out_ref):
    X = xyzT_ref[0]
    Y = xyzT_ref[1]
    Z = xyzT_ref[2]
    shp = (B, NSUB, NLANE)
    idx3 = (jax.lax.broadcasted_iota(jnp.int32, shp, 1) * NLANE
            + jax.lax.broadcasted_iota(jnp.int32, shp, 2))
    asub = jax.lax.broadcasted_iota(jnp.int32, (B, NSUB, M // NSUB), 1)
    alane = jax.lax.broadcasted_iota(jnp.int32, (B, NSUB, M // NSUB), 2)

    def body(i, carry):
        dists, far, cx, cy, cz, acc = carry
        acc = jnp.where((asub == i // (M // NSUB)) & (alane == i % (M // NSUB)),
                        far, acc)
        dx = X - cx
        dy = Y - cy
        dz = Z - cz
        d = dx * dx + dy * dy + dz * dz
        dists = jnp.minimum(dists, d)
        v, ii, tx, ty, tz = dists, idx3, X, Y, Z
        w = NLANE
        while w > 128:
            h = w // 2
            cond = v[..., :h] >= v[..., h:w]
            v = jnp.where(cond, v[..., :h], v[..., h:w])
            ii = jnp.where(cond, ii[..., :h], ii[..., h:w])
            tx = jnp.where(cond, tx[..., :h], tx[..., h:w])
            ty = jnp.where(cond, ty[..., :h], ty[..., h:w])
            tz = jnp.where(cond, tz[..., :h], tz[..., h:w])
            w = h
        mx = jnp.max(v, axis=(1, 2), keepdims=True)
        m2 = v == mx
        far = jnp.min(jnp.where(m2, ii, N), axis=(1, 2), keepdims=True)
        m3 = ii == far
        cx = jnp.sum(jnp.where(m3, tx, 0.0), axis=(1, 2), keepdims=True)
        cy = jnp.sum(jnp.where(m3, ty, 0.0), axis=(1, 2), keepdims=True)
        cz = jnp.sum(jnp.where(m3, tz, 0.0), axis=(1, 2), keepdims=True)
        return dists, far, cx, cy, cz, acc

    dists0 = jnp.full(shp, 1e10, jnp.float32)
    far0 = jnp.zeros((B, 1, 1), jnp.int32)
    acc0 = jnp.zeros((B, NSUB, M // NSUB), jnp.int32)
    cx0 = X[:, 0:1, 0:1]
    cy0 = Y[:, 0:1, 0:1]
    cz0 = Z[:, 0:1, 0:1]
    out = jax.lax.fori_loop(0, M, body,
                            (dists0, far0, cx0, cy0, cz0, acc0))
    out_ref[...] = out[5].reshape(B, M)


@jax.jit
def _fps(xyz):
    xyzT = xyz.transpose(2, 0, 1).reshape(3, B, NSUB, NLANE)
    return pl.pallas_call(
        _fps_kernel,
        out_shape=jax.ShapeDtypeStruct((B, M), jnp.int32),
    )(xyzT)


NT = 2048
F_IN = POS + C_IN


def _pre_kernel(emb_ref, feat_ref, w1a_ref, w1b_ref, b1_ref, out_ref):
    e = emb_ref[0]
    f = feat_ref[0]
    out_ref[0] = (jnp.dot(f, w1b_ref[...], preferred_element_type=jnp.float32)
                  + jnp.dot(e, w1a_ref[...], preferred_element_type=jnp.float32)
                  + b1_ref[0, :])


@jax.jit
def _pre(xyz_embed, features, W1a, W1b, b1):
    return pl.pallas_call(
        _pre_kernel,
        grid=(B, N // NT),
        in_specs=[
            pl.BlockSpec((1, NT, POS), lambda b, i: (b, i, 0)),
            pl.BlockSpec((1, NT, C_IN), lambda b, i: (b, i, 0)),
            pl.BlockSpec((POS, C_OUT), lambda b, i: (0, 0)),
            pl.BlockSpec((C_IN, C_OUT), lambda b, i: (0, 0)),
            pl.BlockSpec((1, C_OUT), lambda b, i: (0, 0)),
        ],
        out_specs=pl.BlockSpec((1, NT, C_OUT), lambda b, i: (b, i, 0)),
        out_shape=jax.ShapeDtypeStruct((B, N, C_OUT), jnp.float32),
    )(xyz_embed, features, W1a, W1b, b1)


TM = 128
NBIN = 64
BINSZ = N // NBIN
ROUNDS = 6
IMAX = 2**31 - 1


def _knn_kernel(xyz_ref, cT_ref, out_ref):
    bno = pl.program_id(0) * N
    x = xyz_ref[0]
    cT = cT_ref[0]
    dot = jnp.dot(x, cT, preferred_element_type=jnp.float32)
    xn2 = jnp.sum(x * x, axis=1, keepdims=True)
    cn2 = jnp.sum(cT * cT, axis=0, keepdims=True)
    d2 = xn2 + cn2 - 2.0 * dot

    bits = jax.lax.bitcast_convert_type(d2, jnp.int32)
    bits3 = bits.reshape(NBIN, BINSZ, TM)
    s7 = jax.lax.broadcasted_iota(jnp.int32, (NBIN, BINSZ, TM), 1)
    P = (bits3 & jnp.int32(~127)) | s7

    cands = []
    for _ in range(ROUNDS):
        mt = jnp.min(P, axis=1)
        cands.append(mt)
        P = jnp.where(P == mt[:, None, :], IMAX, P)
    C0 = jnp.concatenate(cands, axis=0)

    NC = ROUNDS * NBIN
    sC = jax.lax.broadcasted_iota(jnp.int32, (NC, TM), 0)
    kio = jax.lax.broadcasted_iota(jnp.int32, (K, TM), 0)

    def body(k, carry):
        C, outp = carry
        mn = jnp.min(C, axis=0, keepdims=True)
        am = jnp.min(jnp.where(C == mn, sC, IMAX), axis=0,
                     keepdims=True)
        gidx = (((am & (NBIN - 1)) << 7) | (mn & 127)) + bno
        outp = jnp.where(kio == k, gidx, outp)
        C = jnp.where(sC == am, IMAX, C)
        return C, outp

    _, outp = jax.lax.fori_loop(0, K, body,
                                (C0, jnp.zeros((K, TM), jnp.int32)))
    out_ref[0] = outp


@jax.jit
def _knn(xyz, centersT):
    knnT = pl.pallas_call(
        _knn_kernel,
        grid=(B, M // TM),
        in_specs=[
            pl.BlockSpec((1, N, POS), lambda b, i: (b, 0, 0)),
            pl.BlockSpec((1, POS, TM), lambda b, i: (b, 0, i)),
        ],
        out_specs=pl.BlockSpec((1, K, TM), lambda b, i: (b, 0, i)),
        out_shape=jax.ShapeDtypeStruct((B, K, M), jnp.int32),
    )(xyz, centersT)
    return knnT


NWORK = 32
ROWS_TOT = B * M * K
RPW = ROWS_TOT // NWORK
GCH = 128
NCHUNK = RPW // GCH


@functools.lru_cache(maxsize=2)
def _gather_sc_fn(rows_tot):
    rpw = rows_tot // NWORK
    nchunk = rpw // GCH

    @functools.partial(
        pl.kernel,
        mesh=plsc.VectorSubcoreMesh(core_axis_name="c", subcore_axis_name="s"),
        out_type=jax.ShapeDtypeStruct((rows_tot, C_OUT), jnp.float32),
        scratch_types=[
            pltpu.VMEM((nchunk, GCH), jnp.int32),
            pltpu.VMEM((GCH, C_OUT), jnp.float32),
            pltpu.VMEM((GCH, C_OUT), jnp.float32),
            pltpu.SemaphoreType.DMA,
            pltpu.SemaphoreType.DMA,
        ],
    )
    def _gather_sc(table_hbm, gidx_hbm, out_hbm, idx_all, rows0, rows1,
                   s0, s1):
        wid = lax.axis_index("s") * 2 + lax.axis_index("c")
        base = wid * rpw
        pltpu.sync_copy(gidx_hbm.at[wid], idx_all)
        rows = (rows0, rows1)
        sems = (s0, s1)
        pend = pltpu.async_copy(table_hbm.at[idx_all.at[0]], rows0, s0)
        for j in range(1, nchunk + 1):
            nxt = None
            if j < nchunk:
                nxt = pltpu.async_copy(table_hbm.at[idx_all.at[j]],
                                       rows[j % 2], sems[j % 2])
            pend.wait()
            pltpu.sync_copy(rows[(j - 1) % 2],
                            out_hbm.at[pl.ds(base + (j - 1) * GCH, GCH)])
            pend = nxt

    return _gather_sc


TC2 = 32
ROWS = TC2 * K


def _mlp_kernel(g_ref, ce1_ref, w2_ref, b2_ref, out_ref):
    g = g_ref[0]
    h1 = jnp.maximum(g - ce1_ref[0][None, :, :], 0.0).reshape(ROWS, C_OUT)
    h2 = jnp.maximum(
        jnp.dot(h1, w2_ref[...], preferred_element_type=jnp.float32)
        + b2_ref[0, :], 0.0)
    out_ref[0] = jnp.max(h2.reshape(K, TC2, C_OUT), axis=0)


@jax.jit
def _mlp_maxpool(g, ce1, W2, b2):
    nb = g.shape[0]
    return pl.pallas_call(
        _mlp_kernel,
        grid=(nb, M // TC2),
        in_specs=[
            pl.BlockSpec((1, K, TC2, C_OUT), lambda b, i: (b, 0, i, 0)),
            pl.BlockSpec((1, TC2, C_OUT), lambda b, i: (b, i, 0)),
            pl.BlockSpec((C_OUT, C_OUT), lambda b, i: (0, 0)),
            pl.BlockSpec((1, C_OUT), lambda b, i: (0, 0)),
        ],
        out_specs=pl.BlockSpec((1, TC2, C_OUT), lambda b, i: (b, i, 0)),
        out_shape=jax.ShapeDtypeStruct((nb, M, C_OUT), jnp.float32),
    )(g, ce1, W2, b2)


@jax.jit
def kernel(xyz, xyz_embed, features, W1, b1, W2, b2):
    b = xyz.shape[0]
    sample_ids = _fps(jax.lax.stop_gradient(xyz))
    bidx = jnp.arange(b)[:, None]
    centers = xyz[bidx, sample_ids]
    center_embed = xyz_embed[bidx, sample_ids]

    T = _pre(xyz_embed, features, W1[:POS, :], W1[POS:, :],
             b1.reshape(1, C_OUT))

    gidx = _knn(xyz, centers.transpose(0, 2, 1))
    ce1 = (center_embed.reshape(-1, POS) @ W1[:POS, :]).reshape(B, M, C_OUT)

    Tflat = T.reshape(B * N, C_OUT)
    hb = B // 2
    hrows = hb * K * M
    gather = _gather_sc_fn(hrows)
    b2r = b2.reshape(1, C_OUT)
    cfs = []
    for h in range(2):
        gh = gidx[h * hb:(h + 1) * hb].reshape(NWORK, -1, GCH)
        Gh = gather(Tflat, gh)
        cfs.append(_mlp_maxpool(Gh.reshape(hb, K, M, C_OUT),
                                ce1[h * hb:(h + 1) * hb], W2, b2r))
    center_features = jnp.concatenate(cfs, axis=0)
    return (centers, center_embed, center_features, sample_ids)


# speedup vs baseline: 1.1094x; 1.0015x over previous
import functools

import jax
import jax.numpy as jnp
from jax import lax
from jax.experimental import pallas as pl
from jax.experimental.pallas import tpu as pltpu
from jax.experimental.pallas import tpu_sc as plsc

B, N, M, K = 4, 8192, 1024, 32
C_IN, C_OUT, POS = 128, 256, 3

NSUB = 8
NLANE = N // NSUB


def _fps_kernel(xyzT_ref, out_ref):
    X = xyzT_ref[0]
    Y = xyzT_ref[1]
    Z = xyzT_ref[2]
    shp = (B, NSUB, NLANE)
    idx3 = (jax.lax.broadcasted_iota(jnp.int32, shp, 1) * NLANE
            + jax.lax.broadcasted_iota(jnp.int32, shp, 2))
    asub = jax.lax.broadcasted_iota(jnp.int32, (B, NSUB, M // NSUB), 1)
    alane = jax.lax.broadcasted_iota(jnp.int32, (B, NSUB, M // NSUB), 2)

    def body(i, carry):
        dists, far, cx, cy, cz, acc = carry
        acc = jnp.where((asub == i // (M // NSUB)) & (alane == i % (M // NSUB)),
                        far, acc)
        dx = X - cx
        dy = Y - cy
        dz = Z - cz
        d = dx * dx + dy * dy + dz * dz
        dists = jnp.minimum(dists, d)
        v, ii, tx, ty, tz = dists, idx3, X, Y, Z
        w = NLANE
        while w > 128:
            h = w // 2
            cond = v[..., :h] >= v[..., h:w]
            v = jnp.where(cond, v[..., :h], v[..., h:w])
            ii = jnp.where(cond, ii[..., :h], ii[..., h:w])
            tx = jnp.where(cond, tx[..., :h], tx[..., h:w])
            ty = jnp.where(cond, ty[..., :h], ty[..., h:w])
            tz = jnp.where(cond, tz[..., :h], tz[..., h:w])
            w = h
        mx = jnp.max(v, axis=(1, 2), keepdims=True)
        m2 = v == mx
        far = jnp.min(jnp.where(m2, ii, N), axis=(1, 2), keepdims=True)
        m3 = ii == far
        cx = jnp.sum(jnp.where(m3, tx, 0.0), axis=(1, 2), keepdims=True)
        cy = jnp.sum(jnp.where(m3, ty, 0.0), axis=(1, 2), keepdims=True)
        cz = jnp.sum(jnp.where(m3, tz, 0.0), axis=(1, 2), keepdims=True)
        return dists, far, cx, cy, cz, acc

    dists0 = jnp.full(shp, 1e10, jnp.float32)
    far0 = jnp.zeros((B, 1, 1), jnp.int32)
    acc0 = jnp.zeros((B, NSUB, M // NSUB), jnp.int32)
    cx0 = X[:, 0:1, 0:1]
    cy0 = Y[:, 0:1, 0:1]
    cz0 = Z[:, 0:1, 0:1]
    out = jax.lax.fori_loop(0, M, body,
                            (dists0, far0, cx0, cy0, cz0, acc0))
    out_ref[...] = out[5].reshape(B, M)


@jax.jit
def _fps(xyz):
    xyzT = xyz.transpose(2, 0, 1).reshape(3, B, NSUB, NLANE)
    return pl.pallas_call(
        _fps_kernel,
        out_shape=jax.ShapeDtypeStruct((B, M), jnp.int32),
    )(xyzT)


NT = 2048
F_IN = POS + C_IN


def _pre_kernel(emb_ref, feat_ref, w1a_ref, w1b_ref, b1_ref, out_ref):
    e = emb_ref[0]
    f = feat_ref[0]
    out_ref[0] = (jnp.dot(f, w1b_ref[...], preferred_element_type=jnp.float32)
                  + jnp.dot(e, w1a_ref[...], preferred_element_type=jnp.float32)
                  + b1_ref[0, :])


@jax.jit
def _pre(xyz_embed, features, W1a, W1b, b1):
    return pl.pallas_call(
        _pre_kernel,
        grid=(B, N // NT),
        in_specs=[
            pl.BlockSpec((1, NT, POS), lambda b, i: (b, i, 0)),
            pl.BlockSpec((1, NT, C_IN), lambda b, i: (b, i, 0)),
            pl.BlockSpec((POS, C_OUT), lambda b, i: (0, 0)),
            pl.BlockSpec((C_IN, C_OUT), lambda b, i: (0, 0)),
            pl.BlockSpec((1, C_OUT), lambda b, i: (0, 0)),
        ],
        out_specs=pl.BlockSpec((1, NT, C_OUT), lambda b, i: (b, i, 0)),
        out_shape=jax.ShapeDtypeStruct((B, N, C_OUT), jnp.float32),
    )(xyz_embed, features, W1a, W1b, b1)


TM = 128
NBIN = 64
BINSZ = N // NBIN
ROUNDS = 6
IMAX = 2**31 - 1


def _knn_kernel(xyz_ref, cT_ref, out_ref):
    bno = pl.program_id(0) * N
    x = xyz_ref[0]
    cT = cT_ref[0]
    dot = jnp.dot(x, cT, preferred_element_type=jnp.float32)
    xn2 = jnp.sum(x * x, axis=1, keepdims=True)
    cn2 = jnp.sum(cT * cT, axis=0, keepdims=True)
    d2 = xn2 + cn2 - 2.0 * dot

    bits = jax.lax.bitcast_convert_type(d2, jnp.int32)
    bits3 = bits.reshape(NBIN, BINSZ, TM)
    s7 = jax.lax.broadcasted_iota(jnp.int32, (NBIN, BINSZ, TM), 1)
    P = (bits3 & jnp.int32(~127)) | s7

    cands = []
    for _ in range(ROUNDS):
        mt = jnp.min(P, axis=1)
        cands.append(mt)
        P = jnp.where(P == mt[:, None, :], IMAX, P)
    C0 = jnp.concatenate(cands, axis=0)

    NC = ROUNDS * NBIN
    sC = jax.lax.broadcasted_iota(jnp.int32, (NC, TM), 0)
    kio = jax.lax.broadcasted_iota(jnp.int32, (K, TM), 0)

    def body(k, carry):
        C, outp = carry
        mn = jnp.min(C, axis=0, keepdims=True)
        am = jnp.min(jnp.where(C == mn, sC, IMAX), axis=0,
                     keepdims=True)
        gidx = (((am & (NBIN - 1)) << 7) | (mn & 127)) + bno
        outp = jnp.where(kio == k, gidx, outp)
        C = jnp.where(sC == am, IMAX, C)
        return C, outp

    _, outp = jax.lax.fori_loop(0, K, body,
                                (C0, jnp.zeros((K, TM), jnp.int32)))
    out_ref[0] = outp


@jax.jit
def _knn(xyz, centersT):
    knnT = pl.pallas_call(
        _knn_kernel,
        grid=(B, M // TM),
        in_specs=[
            pl.BlockSpec((1, N, POS), lambda b, i: (b, 0, 0)),
            pl.BlockSpec((1, POS, TM), lambda b, i: (b, 0, i)),
        ],
        out_specs=pl.BlockSpec((1, K, TM), lambda b, i: (b, 0, i)),
        out_shape=jax.ShapeDtypeStruct((B, K, M), jnp.int32),
    )(xyz, centersT)
    return knnT


NWORK = 32
ROWS_TOT = B * M * K
RPW = ROWS_TOT // NWORK
GCH = 128
NCHUNK = RPW // GCH


@functools.lru_cache(maxsize=2)
def _gather_sc_fn(rows_tot):
    rpw = rows_tot // NWORK
    nchunk = rpw // GCH

    @functools.partial(
        pl.kernel,
        mesh=plsc.VectorSubcoreMesh(core_axis_name="c", subcore_axis_name="s"),
        out_type=jax.ShapeDtypeStruct((rows_tot, C_OUT), jnp.float32),
        scratch_types=[
            pltpu.VMEM((nchunk, GCH), jnp.int32),
            pltpu.VMEM((GCH, C_OUT), jnp.float32),
            pltpu.VMEM((GCH, C_OUT), jnp.float32),
            pltpu.SemaphoreType.DMA,
            pltpu.SemaphoreType.DMA,
        ],
    )
    def _gather_sc(table_hbm, gidx_hbm, out_hbm, idx_all, rows0, rows1,
                   s0, s1):
        wid = lax.axis_index("s") * 2 + lax.axis_index("c")
        base = wid * rpw
        pltpu.sync_copy(gidx_hbm.at[wid], idx_all)
        rows = (rows0, rows1)
        sems = (s0, s1)
        pend = pltpu.async_copy(table_hbm.at[idx_all.at[0]], rows0, s0)
        for j in range(1, nchunk + 1):
            nxt = None
            if j < nchunk:
                nxt = pltpu.async_copy(table_hbm.at[idx_all.at[j]],
                                       rows[j % 2], sems[j % 2])
            pend.wait()
            pltpu.sync_copy(rows[(j - 1) % 2],
                            out_hbm.at[pl.ds(base + (j - 1) * GCH, GCH)])
            pend = nxt

    return _gather_sc


TC2 = 32
ROWS = TC2 * K


def _mlp_kernel(g_ref, ce_ref, w1a_ref, w2_ref, b2_ref, out_ref):
    g = g_ref[0]
    ce1 = jnp.dot(ce_ref[0], w1a_ref[...],
                  preferred_element_type=jnp.float32)
    h1 = jnp.maximum(g - ce1[None, :, :], 0.0).reshape(ROWS, C_OUT)
    h2 = jnp.maximum(
        jnp.dot(h1, w2_ref[...], preferred_element_type=jnp.float32)
        + b2_ref[0, :], 0.0)
    out_ref[0] = jnp.max(h2.reshape(K, TC2, C_OUT), axis=0)


@jax.jit
def _mlp_maxpool(g, center_embed, W1a, W2, b2):
    nb = g.shape[0]
    return pl.pallas_call(
        _mlp_kernel,
        grid=(nb, M // TC2),
        in_specs=[
            pl.BlockSpec((1, K, TC2, C_OUT), lambda b, i: (b, 0, i, 0)),
            pl.BlockSpec((1, TC2, POS), lambda b, i: (b, i, 0)),
            pl.BlockSpec((POS, C_OUT), lambda b, i: (0, 0)),
            pl.BlockSpec((C_OUT, C_OUT), lambda b, i: (0, 0)),
            pl.BlockSpec((1, C_OUT), lambda b, i: (0, 0)),
        ],
        out_specs=pl.BlockSpec((1, TC2, C_OUT), lambda b, i: (b, i, 0)),
        out_shape=jax.ShapeDtypeStruct((nb, M, C_OUT), jnp.float32),
    )(g, center_embed, W1a, W2, b2)


@jax.jit
def kernel(xyz, xyz_embed, features, W1, b1, W2, b2):
    b = xyz.shape[0]
    sample_ids = _fps(jax.lax.stop_gradient(xyz))
    bidx = jnp.arange(b)[:, None]
    centers = xyz[bidx, sample_ids]
    center_embed = xyz_embed[bidx, sample_ids]

    T = _pre(xyz_embed, features, W1[:POS, :], W1[POS:, :],
             b1.reshape(1, C_OUT))

    gidx = _knn(xyz, centers.transpose(0, 2, 1))

    Tflat = T.reshape(B * N, C_OUT)
    hb = B // 2
    hrows = hb * K * M
    gather = _gather_sc_fn(hrows)
    b2r = b2.reshape(1, C_OUT)
    W1a = W1[:POS, :]
    cfs = []
    for h in range(2):
        gh = gidx[h * hb:(h + 1) * hb].reshape(NWORK, -1, GCH)
        Gh = gather(Tflat, gh)
        cfs.append(_mlp_maxpool(Gh.reshape(hb, K, M, C_OUT),
                                center_embed[h * hb:(h + 1) * hb], W1a,
                                W2, b2r))
    center_features = jnp.concatenate(cfs, axis=0)
    return (centers, center_embed, center_features, sample_ids)
